# baseline (device time: 86773 ns/iter reference)
import jax
import jax.numpy as jnp
from jax import lax
from jax.experimental import pallas as pl
from jax.experimental.pallas import tpu as pltpu

N_DEV = 4
B, SQ, SKV, D = 4, 256, 1024, 1024
H, DH = 8, 128
SCALE = 0.08838834764831843
CH = SQ


def _attn_chunk(b, x_ref, k_ref, v_ref, wq, wo):
    xb = x_ref[b].astype(jnp.bfloat16)
    q = jnp.dot(xb, wq, preferred_element_type=jnp.float32).astype(
        jnp.bfloat16
    )
    o_parts = []
    for h in range(H):
        qh = q[:, h * DH:(h + 1) * DH]
        kh = k_ref[b * SKV:(b + 1) * SKV, h * DH:(h + 1) * DH]
        s = lax.dot_general(
            qh, kh, (((1,), (1,)), ((), ())),
            preferred_element_type=jnp.float32,
        ) * SCALE
        p = jnp.exp(s)
        l = jnp.sum(p, axis=1, keepdims=True)
        vh = v_ref[b * SKV:(b + 1) * SKV, h * DH:(h + 1) * DH]
        o = lax.dot_general(
            p.astype(jnp.bfloat16), vh, (((1,), (0,)), ((), ())),
            preferred_element_type=jnp.float32,
        )
        o_parts.append((o / l).astype(jnp.bfloat16))
    o_b = jnp.concatenate(o_parts, axis=1)
    return jnp.dot(o_b, wo, preferred_element_type=jnp.float32)


def _body(x_ref, wq_ref, k_ref, v_ref, wo_ref, out_ref,
          stage_ref, send1, recv1, send2, recv2):
    my = lax.axis_index("i")

    barrier = pltpu.get_barrier_semaphore()
    for off in (1, 2, 3):
        pl.semaphore_signal(
            barrier, inc=1, device_id=((my + off) % N_DEV,),
            device_id_type=pl.DeviceIdType.MESH,
        )
    pl.semaphore_wait(barrier, 3)

    wq = wq_ref[...].astype(jnp.bfloat16)
    wo = wo_ref[...].astype(jnp.bfloat16)

    for step in range(N_DEV):
        for b in range(B):
            @pl.when(my == (b - step - 1) % N_DEV)
            def _compute(b=b, step=step):
                chunk = _attn_chunk(b, x_ref, k_ref, v_ref, wq, wo)
                out_ref[b * CH:(b + 1) * CH, :] = chunk.astype(jnp.bfloat16)
                if step < N_DEV - 1:
                    rdma = pltpu.make_async_remote_copy(
                        src_ref=out_ref.at[b * CH:(b + 1) * CH],
                        dst_ref=stage_ref.at[my],
                        send_sem=send1.at[b],
                        recv_sem=recv1.at[my],
                        device_id=(b,),
                        device_id_type=pl.DeviceIdType.MESH,
                    )
                    rdma.start()

    own = out_ref[pl.ds(my * CH, CH), :].astype(jnp.float32)
    for off in (1, 2, 3):
        k = (my + off) % N_DEV
        recv = pltpu.make_async_remote_copy(
            src_ref=stage_ref.at[k],
            dst_ref=stage_ref.at[k],
            send_sem=send1.at[k],
            recv_sem=recv1.at[k],
            device_id=(k,),
            device_id_type=pl.DeviceIdType.MESH,
        )
        recv.wait_recv()
        own = own + stage_ref[k].astype(jnp.float32)
    out_ref[pl.ds(my * CH, CH), :] = own.astype(jnp.bfloat16)

    for off in (1, 2, 3):
        dest = (my + off) % N_DEV
        rdma = pltpu.make_async_remote_copy(
            src_ref=out_ref.at[pl.ds(my * CH, CH)],
            dst_ref=out_ref.at[pl.ds(my * CH, CH)],
            send_sem=send2.at[dest],
            recv_sem=recv2.at[my],
            device_id=(dest,),
            device_id_type=pl.DeviceIdType.MESH,
        )
        rdma.start()

    for off in (1, 2, 3):
        k = (my + off) % N_DEV
        recv = pltpu.make_async_remote_copy(
            src_ref=out_ref.at[pl.ds(k * CH, CH)],
            dst_ref=out_ref.at[pl.ds(k * CH, CH)],
            send_sem=send2.at[k],
            recv_sem=recv2.at[k],
            device_id=(k,),
            device_id_type=pl.DeviceIdType.MESH,
        )
        recv.wait_recv()

    for off in (1, 2, 3):
        dest = (my + off) % N_DEV
        for sems in (send1, send2):
            drain = pltpu.make_async_remote_copy(
                src_ref=out_ref.at[pl.ds(my * CH, CH)],
                dst_ref=out_ref.at[pl.ds(my * CH, CH)],
                send_sem=sems.at[dest],
                recv_sem=recv2.at[my],
                device_id=(dest,),
                device_id_type=pl.DeviceIdType.MESH,
            )
            drain.wait_send()


def kernel(x, Wq, Wo, K_ext, V_ext):
    bf16 = jnp.bfloat16
    k2 = K_ext.reshape(B * SKV, H * DH).astype(bf16)
    v2 = V_ext.reshape(B * SKV, H * DH).astype(bf16)

    out = pl.pallas_call(
        _body,
        out_shape=jax.ShapeDtypeStruct((B * SQ, D), bf16),
        in_specs=[pl.BlockSpec(memory_space=pltpu.VMEM)] * 5,
        out_specs=pl.BlockSpec(memory_space=pltpu.VMEM),
        scratch_shapes=[
            pltpu.VMEM((N_DEV, CH, D), bf16),
            pltpu.SemaphoreType.DMA((N_DEV,)),
            pltpu.SemaphoreType.DMA((N_DEV,)),
            pltpu.SemaphoreType.DMA((N_DEV,)),
            pltpu.SemaphoreType.DMA((N_DEV,)),
        ],
        compiler_params=pltpu.CompilerParams(
            collective_id=0,
            vmem_limit_bytes=100 * 1024 * 1024,
        ),
    )(x, Wq, k2, v2, Wo)

    return out.reshape(B, SQ, D)


# device time: 78201 ns/iter; 1.1096x vs baseline; 1.1096x over previous
import jax
import jax.numpy as jnp
from jax import lax
from jax.experimental import pallas as pl
from jax.experimental.pallas import tpu as pltpu

N_DEV = 4
B, SQ, SKV, D = 4, 256, 1024, 1024
H, DH = 8, 128
SCALE = 0.08838834764831843
CH = SQ


def _attn_chunk(b, x_ref, k_ref, v_ref, wq, wo):
    xb = x_ref[b].astype(jnp.bfloat16)
    q = jnp.dot(xb, wq, preferred_element_type=jnp.float32).astype(
        jnp.bfloat16
    )
    o_parts = []
    for h in range(H):
        qh = q[:, h * DH:(h + 1) * DH]
        kh = k_ref[b * SKV:(b + 1) * SKV, h * DH:(h + 1) * DH]
        s = lax.dot_general(
            qh, kh, (((1,), (1,)), ((), ())),
            preferred_element_type=jnp.float32,
        ) * SCALE
        p = jnp.exp(s)
        l = jnp.sum(p, axis=1, keepdims=True)
        vh = v_ref[b * SKV:(b + 1) * SKV, h * DH:(h + 1) * DH]
        o = lax.dot_general(
            p.astype(jnp.bfloat16), vh, (((1,), (0,)), ((), ())),
            preferred_element_type=jnp.float32,
        )
        o_parts.append((o / l).astype(jnp.bfloat16))
    o_b = jnp.concatenate(o_parts, axis=1)
    return jnp.dot(o_b, wo, preferred_element_type=jnp.float32)


def _body(x_ref, wq_ref, k_ref, v_ref, wo_ref, out_ref,
          stage_ref, send1, recv1, send2, recv2):
    my = lax.axis_index("i")

    barrier = pltpu.get_barrier_semaphore()
    for off in (1, 2, 3):
        pl.semaphore_signal(
            barrier, inc=1, device_id=((my + off) % N_DEV,),
            device_id_type=pl.DeviceIdType.MESH,
        )
    pl.semaphore_wait(barrier, 3)

    wq = wq_ref[...].astype(jnp.bfloat16)
    wo = wo_ref[...].astype(jnp.bfloat16)

    for b in range(B):
        chunk = _attn_chunk(b, x_ref, k_ref, v_ref, wq, wo)
        out_ref[b * CH:(b + 1) * CH, :] = chunk.astype(jnp.bfloat16)

        @pl.when(my != b)
        def _send(b=b):
            rdma = pltpu.make_async_remote_copy(
                src_ref=out_ref.at[b * CH:(b + 1) * CH],
                dst_ref=stage_ref.at[my],
                send_sem=send1.at[b],
                recv_sem=recv1.at[my],
                device_id=(b,),
                device_id_type=pl.DeviceIdType.MESH,
            )
            rdma.start()

    own = out_ref[pl.ds(my * CH, CH), :].astype(jnp.float32)
    for off in (1, 2, 3):
        k = (my + off) % N_DEV
        recv = pltpu.make_async_remote_copy(
            src_ref=stage_ref.at[k],
            dst_ref=stage_ref.at[k],
            send_sem=send1.at[k],
            recv_sem=recv1.at[k],
            device_id=(k,),
            device_id_type=pl.DeviceIdType.MESH,
        )
        recv.wait_recv()
        own = own + stage_ref[k].astype(jnp.float32)
    out_ref[pl.ds(my * CH, CH), :] = own.astype(jnp.bfloat16)

    for off in (1, 2, 3):
        dest = (my + off) % N_DEV
        rdma = pltpu.make_async_remote_copy(
            src_ref=out_ref.at[pl.ds(my * CH, CH)],
            dst_ref=out_ref.at[pl.ds(my * CH, CH)],
            send_sem=send2.at[dest],
            recv_sem=recv2.at[my],
            device_id=(dest,),
            device_id_type=pl.DeviceIdType.MESH,
        )
        rdma.start()

    for off in (1, 2, 3):
        k = (my + off) % N_DEV
        recv = pltpu.make_async_remote_copy(
            src_ref=out_ref.at[pl.ds(k * CH, CH)],
            dst_ref=out_ref.at[pl.ds(k * CH, CH)],
            send_sem=send2.at[k],
            recv_sem=recv2.at[k],
            device_id=(k,),
            device_id_type=pl.DeviceIdType.MESH,
        )
        recv.wait_recv()

    for off in (1, 2, 3):
        dest = (my + off) % N_DEV
        for sems in (send1, send2):
            drain = pltpu.make_async_remote_copy(
                src_ref=out_ref.at[pl.ds(my * CH, CH)],
                dst_ref=out_ref.at[pl.ds(my * CH, CH)],
                send_sem=sems.at[dest],
                recv_sem=recv2.at[my],
                device_id=(dest,),
                device_id_type=pl.DeviceIdType.MESH,
            )
            drain.wait_send()


def kernel(x, Wq, Wo, K_ext, V_ext):
    bf16 = jnp.bfloat16
    k2 = K_ext.reshape(B * SKV, H * DH).astype(bf16)
    v2 = V_ext.reshape(B * SKV, H * DH).astype(bf16)

    out = pl.pallas_call(
        _body,
        out_shape=jax.ShapeDtypeStruct((B * SQ, D), bf16),
        in_specs=[pl.BlockSpec(memory_space=pltpu.VMEM)] * 5,
        out_specs=pl.BlockSpec(memory_space=pltpu.VMEM),
        scratch_shapes=[
            pltpu.VMEM((N_DEV, CH, D), bf16),
            pltpu.SemaphoreType.DMA((N_DEV,)),
            pltpu.SemaphoreType.DMA((N_DEV,)),
            pltpu.SemaphoreType.DMA((N_DEV,)),
            pltpu.SemaphoreType.DMA((N_DEV,)),
        ],
        compiler_params=pltpu.CompilerParams(
            collective_id=0,
            vmem_limit_bytes=100 * 1024 * 1024,
        ),
    )(x, Wq, k2, v2, Wo)

    return out.reshape(B, SQ, D)
